# baseline (speedup 1.0000x reference)
import functools

import jax
import jax.numpy as jnp
from jax import lax
from jax.experimental import pallas as pl
from jax.experimental.pallas import tpu as pltpu
from jax.experimental.pallas import tpu_sc as plsc

T = 8
N = 512
F = 128
H1 = 64
H2 = 8
E = 8192
NH = N * H2

_NC = 2
_NS = 16
_TS_PER_CORE = T // _NC
_CHUNKS = _NS // _TS_PER_CORE
_EC = E // _CHUNKS
_A_WORDS = N * N
_SH_WORDS = _TS_PER_CORE * _A_WORDS
_ZW = _SH_WORDS // _NS
_SCAT = 128
_NSCAT = _EC // _SCAT
_NIDX = _EC // 16


_ZB = 8192


def _adj_body(edges, weights, out, src_v, dst_v, w_v, idx2, w2, zbuf, a_sh):
    c = lax.axis_index("c")
    s = lax.axis_index("s")
    ts_local = s // _CHUNKS
    ts = c * _TS_PER_CORE + ts_local
    chunk = s % _CHUNKS

    def zfill(i, _):
        zbuf[pl.ds(i * 16, 16)] = jnp.zeros((16,), jnp.float32)
        return 0

    lax.fori_loop(0, _ZB // 16, zfill, 0)

    def zcopy(i, _):
        pltpu.sync_copy(zbuf, a_sh.at[pl.ds(s * _ZW + i * _ZB, _ZB)])
        return 0

    lax.fori_loop(0, _ZW // _ZB, zcopy, 0)
    pltpu.sync_copy(edges.at[ts, 0, pl.ds(chunk * _EC, _EC)], src_v)
    pltpu.sync_copy(edges.at[ts, 1, pl.ds(chunk * _EC, _EC)], dst_v)
    pltpu.sync_copy(weights.at[ts, pl.ds(chunk * _EC, _EC)], w_v)

    base = ts_local * _A_WORDS

    def idx_body(i, _):
        d = dst_v[pl.ds(i * 16, 16)]
        sr = src_v[pl.ds(i * 16, 16)]
        ww = w_v[pl.ds(i * 16, 16)]
        ix = base + d * N + sr
        row = i // (_SCAT // 16)
        col = (i % (_SCAT // 16)) * 16
        idx2[row, pl.ds(col, 16)] = ix
        w2[row, pl.ds(col, 16)] = ww
        return 0

    lax.fori_loop(0, _NIDX, idx_body, 0)
    plsc.subcore_barrier()

    def scat_body(j, _):
        pltpu.sync_copy(w2.at[j], a_sh.at[idx2.at[j]], add=True)
        return 0

    lax.fori_loop(0, _NSCAT, scat_body, 0)
    plsc.subcore_barrier()

    out_row = c * _TS_PER_CORE + s // _CHUNKS
    out_col = (s % _CHUNKS) * _ZW
    pltpu.sync_copy(a_sh.at[pl.ds(s * _ZW, _ZW)],
                    out.at[out_row, pl.ds(out_col, _ZW)])


def _build_adj(edge_index, edge_weight):
    mesh = plsc.VectorSubcoreMesh(core_axis_name="c", subcore_axis_name="s")
    k = pl.kernel(
        _adj_body,
        mesh=mesh,
        out_type=jax.ShapeDtypeStruct((T, _A_WORDS), jnp.float32),
        scratch_types=[
            pltpu.VMEM((_EC,), jnp.int32),
            pltpu.VMEM((_EC,), jnp.int32),
            pltpu.VMEM((_EC,), jnp.float32),
            pltpu.VMEM((_NSCAT, _SCAT), jnp.int32),
            pltpu.VMEM((_NSCAT, _SCAT), jnp.float32),
            pltpu.VMEM((_ZB,), jnp.float32),
            pltpu.VMEM_SHARED((_SH_WORDS,), jnp.float32),
        ],
    )
    return k(edge_index, edge_weight)


def _xw1_body(x_ref, w1_ref, o_ref):
    o_ref[0] = jnp.dot(x_ref[0], w1_ref[0], preferred_element_type=jnp.float32)


def _xw1(x, W1):
    return pl.pallas_call(
        _xw1_body,
        grid=(T,),
        in_specs=[
            pl.BlockSpec((1, N, F), lambda i: (i, 0, 0)),
            pl.BlockSpec((1, F, H1), lambda i: (i, 0, 0)),
        ],
        out_specs=pl.BlockSpec((1, N, H1), lambda i: (i, 0, 0)),
        out_shape=jax.ShapeDtypeStruct((T, N, H1), jnp.float32),
    )(x, W1)


def _gcn_body(a_ref, xw1_ref, n_ref, w2_ref, z_ref):
    a = a_ref[0]
    h = jnp.maximum(jnp.dot(a, xw1_ref[0], preferred_element_type=jnp.float32),
                    0.0)
    h = h + 0.1 * n_ref[0]
    hw2 = jnp.dot(h, w2_ref[0], preferred_element_type=jnp.float32)
    z_ref[0] = jnp.dot(a, hw2, preferred_element_type=jnp.float32)


def _gcn(adj, xw1, noise, W2):
    return pl.pallas_call(
        _gcn_body,
        grid=(T,),
        in_specs=[
            pl.BlockSpec((1, N, N), lambda i: (i, 0, 0)),
            pl.BlockSpec((1, N, H1), lambda i: (i, 0, 0)),
            pl.BlockSpec((1, N, H1), lambda i: (i, 0, 0)),
            pl.BlockSpec((1, H1, H2), lambda i: (i, 0, 0)),
        ],
        out_specs=pl.BlockSpec((1, N, H2), lambda i: (i, 0, 0)),
        out_shape=jax.ShapeDtypeStruct((T, N, H2), jnp.float32),
    )(adj, xw1, noise, W2)


_KB = 512
_NKB = 2 * NH // _KB


def _shifted(x, shift):
    return jnp.concatenate(
        [jnp.zeros((shift, NH), jnp.float32), x[:T - shift]], 0)


def _tcn_body(s_ref, w0_ref, w1_ref, b0_ref, b1_ref, y1_ref, x2_scr, acc):
    st = pl.program_id(0)
    k = pl.program_id(1)

    @pl.when((st == 0) & (k == 0))
    def _init0():
        s = s_ref[...]
        x2_scr[:, pl.ds(0, NH)] = _shifted(s, 1)
        x2_scr[:, pl.ds(NH, NH)] = s
        acc[...] = jnp.broadcast_to(b0_ref[...], (T, NH))

    @pl.when(st == 0)
    def _acc0():
        xb = x2_scr[:, pl.ds(k * _KB, _KB)]
        acc[...] += jnp.dot(xb, w0_ref[...],
                            preferred_element_type=jnp.float32)

    @pl.when(st == 1)
    def _acc1():
        xb = x2_scr[:, pl.ds(k * _KB, _KB)]
        acc[...] += jnp.dot(xb, w1_ref[...],
                            preferred_element_type=jnp.float32)

    @pl.when((st == 0) & (k == _NKB - 1))
    def _fin0():
        y0 = jnp.maximum(jnp.maximum(acc[...], 0.0) + s_ref[...], 0.0)
        x2_scr[:, pl.ds(0, NH)] = _shifted(y0, 2)
        x2_scr[:, pl.ds(NH, NH)] = y0
        acc[...] = jnp.broadcast_to(b1_ref[...], (T, NH))

    @pl.when((st == 1) & (k == _NKB - 1))
    def _fin1():
        y0 = x2_scr[:, pl.ds(NH, NH)]
        y1_ref[...] = jnp.maximum(jnp.maximum(acc[...], 0.0) + y0, 0.0)


def _tcn(s, w0_cat, w1_cat, b0, b1):
    return pl.pallas_call(
        _tcn_body,
        grid=(2, _NKB),
        in_specs=[
            pl.BlockSpec((T, NH), lambda st, k: (0, 0)),
            pl.BlockSpec((_KB, NH),
                         lambda st, k: (jnp.where(st == 0, k, _NKB - 1), 0)),
            pl.BlockSpec((_KB, NH),
                         lambda st, k: (jnp.where(st == 0, 0, k), 0)),
            pl.BlockSpec((1, NH), lambda st, k: (0, 0)),
            pl.BlockSpec((1, NH), lambda st, k: (0, 0)),
        ],
        out_specs=pl.BlockSpec((T, NH), lambda st, k: (0, 0)),
        out_shape=jax.ShapeDtypeStruct((T, NH), jnp.float32),
        scratch_shapes=[
            pltpu.VMEM((T, 2 * NH), jnp.float32),
            pltpu.VMEM((T, NH), jnp.float32),
        ],
    )(s, w0_cat, w1_cat, b0, b1)


def _outer_body(z_ref, y_ref, r_ref):
    z = z_ref[0]
    y = y_ref[0]
    r_ref[0, 0] = lax.dot_general(z, z, (((1,), (1,)), ((), ())),
                                  preferred_element_type=jnp.float32)
    r_ref[1, 0] = lax.dot_general(y, y, (((1,), (1,)), ((), ())),
                                  preferred_element_type=jnp.float32)


def _outer(z, y1):
    return pl.pallas_call(
        _outer_body,
        grid=(T,),
        in_specs=[
            pl.BlockSpec((1, N, H2), lambda i: (i, 0, 0)),
            pl.BlockSpec((1, N, H2), lambda i: (i, 0, 0)),
        ],
        out_specs=pl.BlockSpec((2, 1, N, N), lambda i: (0, i, 0, 0)),
        out_shape=jax.ShapeDtypeStruct((2, T, N, N), jnp.float32),
    )(z, y1)


def kernel(struct_features, edge_index, edge_weight, noise, W1, W2,
           tcn_w0, tcn_b0, tcn_w1, tcn_b1):
    edge_index = edge_index.astype(jnp.int32)
    edge_weight = edge_weight.astype(jnp.float32)

    adj = _build_adj(edge_index, edge_weight).reshape(T, N, N)

    xw1 = _xw1(struct_features, W1)
    z = _gcn(adj, xw1, noise, W2)
    s = z.reshape(T, NH)

    y1 = _tcn(s, tcn_w0.reshape(2 * NH, NH), tcn_w1.reshape(2 * NH, NH),
              tcn_b0.reshape(1, NH), tcn_b1.reshape(1, NH))

    out = _outer(z, y1.reshape(T, N, H2))
    return out.reshape(2, T, N * N)


# speedup vs baseline: 1.0139x; 1.0139x over previous
import functools

import jax
import jax.numpy as jnp
from jax import lax
from jax.experimental import pallas as pl
from jax.experimental.pallas import tpu as pltpu
from jax.experimental.pallas import tpu_sc as plsc

T = 8
N = 512
F = 128
H1 = 64
H2 = 8
E = 8192
NH = N * H2

_NC = 2
_NS = 16
_TS_PER_CORE = T // _NC
_CHUNKS = _NS // _TS_PER_CORE
_EC = E // _CHUNKS
_A_WORDS = N * N
_SH_WORDS = _TS_PER_CORE * _A_WORDS
_ZW = _SH_WORDS // _NS
_SCAT = 128
_NSCAT = _EC // _SCAT
_NIDX = _EC // 16


_ZB = 8192


def _adj_body(edges, weights, out, src_v, dst_v, w_v, idx2, w2, zbuf, a_sh):
    c = lax.axis_index("c")
    s = lax.axis_index("s")
    ts_local = s // _CHUNKS
    ts = c * _TS_PER_CORE + ts_local
    chunk = s % _CHUNKS

    def zfill(i, _):
        zbuf[pl.ds(i * 16, 16)] = jnp.zeros((16,), jnp.float32)
        return 0

    lax.fori_loop(0, _ZB // 16, zfill, 0)

    def zcopy(i, _):
        pltpu.sync_copy(zbuf, a_sh.at[pl.ds(s * _ZW + i * _ZB, _ZB)])
        return 0

    lax.fori_loop(0, _ZW // _ZB, zcopy, 0)
    pltpu.sync_copy(edges.at[ts, 0, pl.ds(chunk * _EC, _EC)], src_v)
    pltpu.sync_copy(edges.at[ts, 1, pl.ds(chunk * _EC, _EC)], dst_v)
    pltpu.sync_copy(weights.at[ts, pl.ds(chunk * _EC, _EC)], w_v)

    base = ts_local * _A_WORDS

    def idx_body(i, _):
        d = dst_v[pl.ds(i * 16, 16)]
        sr = src_v[pl.ds(i * 16, 16)]
        ww = w_v[pl.ds(i * 16, 16)]
        ix = base + d * N + sr
        row = i // (_SCAT // 16)
        col = (i % (_SCAT // 16)) * 16
        idx2[row, pl.ds(col, 16)] = ix
        w2[row, pl.ds(col, 16)] = ww
        return 0

    lax.fori_loop(0, _NIDX, idx_body, 0)
    plsc.subcore_barrier()

    def scat_body(j, _):
        pltpu.sync_copy(w2.at[j], a_sh.at[idx2.at[j]], add=True)
        return 0

    lax.fori_loop(0, _NSCAT, scat_body, 0)
    plsc.subcore_barrier()

    out_row = c * _TS_PER_CORE + s // _CHUNKS
    out_col = (s % _CHUNKS) * _ZW
    pltpu.sync_copy(a_sh.at[pl.ds(s * _ZW, _ZW)],
                    out.at[out_row, pl.ds(out_col, _ZW)])


def _build_adj(edge_index, edge_weight):
    mesh = plsc.VectorSubcoreMesh(core_axis_name="c", subcore_axis_name="s")
    k = pl.kernel(
        _adj_body,
        mesh=mesh,
        out_type=jax.ShapeDtypeStruct((T, _A_WORDS), jnp.float32),
        scratch_types=[
            pltpu.VMEM((_EC,), jnp.int32),
            pltpu.VMEM((_EC,), jnp.int32),
            pltpu.VMEM((_EC,), jnp.float32),
            pltpu.VMEM((_NSCAT, _SCAT), jnp.int32),
            pltpu.VMEM((_NSCAT, _SCAT), jnp.float32),
            pltpu.VMEM((_ZB,), jnp.float32),
            pltpu.VMEM_SHARED((_SH_WORDS,), jnp.float32),
        ],
    )
    return k(edge_index, edge_weight)


def _xw1_body(x_ref, w1_ref, o_ref):
    o_ref[0] = jnp.dot(x_ref[0], w1_ref[0], preferred_element_type=jnp.float32)


def _xw1(x, W1):
    return pl.pallas_call(
        _xw1_body,
        grid=(T,),
        in_specs=[
            pl.BlockSpec((1, N, F), lambda i: (i, 0, 0)),
            pl.BlockSpec((1, F, H1), lambda i: (i, 0, 0)),
        ],
        out_specs=pl.BlockSpec((1, N, H1), lambda i: (i, 0, 0)),
        out_shape=jax.ShapeDtypeStruct((T, N, H1), jnp.float32),
    )(x, W1)


def _gcn_body(a_ref, x_ref, w1_ref, n_ref, w2_ref, z_ref):
    a = a_ref[0]
    xw1 = jnp.dot(x_ref[0], w1_ref[0], preferred_element_type=jnp.float32)
    h = jnp.maximum(jnp.dot(a, xw1, preferred_element_type=jnp.float32),
                    0.0)
    h = h + 0.1 * n_ref[0]
    hw2 = jnp.dot(h, w2_ref[0], preferred_element_type=jnp.float32)
    z_ref[0] = jnp.dot(a, hw2, preferred_element_type=jnp.float32)


def _gcn(adj, x, W1, noise, W2):
    return pl.pallas_call(
        _gcn_body,
        grid=(T,),
        in_specs=[
            pl.BlockSpec((1, N, N), lambda i: (i, 0, 0)),
            pl.BlockSpec((1, N, F), lambda i: (i, 0, 0)),
            pl.BlockSpec((1, F, H1), lambda i: (i, 0, 0)),
            pl.BlockSpec((1, N, H1), lambda i: (i, 0, 0)),
            pl.BlockSpec((1, H1, H2), lambda i: (i, 0, 0)),
        ],
        out_specs=pl.BlockSpec((1, N, H2), lambda i: (i, 0, 0)),
        out_shape=jax.ShapeDtypeStruct((T, N, H2), jnp.float32),
    )(adj, x, W1, noise, W2)


_KB = 512
_NKB = 2 * NH // _KB


def _shifted(x, shift):
    return jnp.concatenate(
        [jnp.zeros((shift, NH), jnp.float32), x[:T - shift]], 0)


def _tcn_body(s_ref, w0_ref, w1_ref, b0_ref, b1_ref, y1_ref, x2_scr, acc):
    st = pl.program_id(0)
    k = pl.program_id(1)

    @pl.when((st == 0) & (k == 0))
    def _init0():
        s = s_ref[...]
        x2_scr[:, pl.ds(0, NH)] = _shifted(s, 1)
        x2_scr[:, pl.ds(NH, NH)] = s
        acc[...] = jnp.broadcast_to(b0_ref[...], (T, NH))

    @pl.when(st == 0)
    def _acc0():
        xb = x2_scr[:, pl.ds(k * _KB, _KB)]
        acc[...] += jnp.dot(xb, w0_ref[...],
                            preferred_element_type=jnp.float32)

    @pl.when(st == 1)
    def _acc1():
        xb = x2_scr[:, pl.ds(k * _KB, _KB)]
        acc[...] += jnp.dot(xb, w1_ref[...],
                            preferred_element_type=jnp.float32)

    @pl.when((st == 0) & (k == _NKB - 1))
    def _fin0():
        y0 = jnp.maximum(jnp.maximum(acc[...], 0.0) + s_ref[...], 0.0)
        x2_scr[:, pl.ds(0, NH)] = _shifted(y0, 2)
        x2_scr[:, pl.ds(NH, NH)] = y0
        acc[...] = jnp.broadcast_to(b1_ref[...], (T, NH))

    @pl.when((st == 1) & (k == _NKB - 1))
    def _fin1():
        y0 = x2_scr[:, pl.ds(NH, NH)]
        y1_ref[...] = jnp.maximum(jnp.maximum(acc[...], 0.0) + y0, 0.0)


def _tcn(s, w0_cat, w1_cat, b0, b1):
    return pl.pallas_call(
        _tcn_body,
        grid=(2, _NKB),
        in_specs=[
            pl.BlockSpec((T, NH), lambda st, k: (0, 0)),
            pl.BlockSpec((_KB, NH),
                         lambda st, k: (jnp.where(st == 0, k, _NKB - 1), 0)),
            pl.BlockSpec((_KB, NH),
                         lambda st, k: (jnp.where(st == 0, 0, k), 0)),
            pl.BlockSpec((1, NH), lambda st, k: (0, 0)),
            pl.BlockSpec((1, NH), lambda st, k: (0, 0)),
        ],
        out_specs=pl.BlockSpec((T, NH), lambda st, k: (0, 0)),
        out_shape=jax.ShapeDtypeStruct((T, NH), jnp.float32),
        scratch_shapes=[
            pltpu.VMEM((T, 2 * NH), jnp.float32),
            pltpu.VMEM((T, NH), jnp.float32),
        ],
    )(s, w0_cat, w1_cat, b0, b1)


def _outer_body(z_ref, y_ref, r_ref):
    z = z_ref[0]
    y = y_ref[0]
    r_ref[0, 0] = lax.dot_general(z, z, (((1,), (1,)), ((), ())),
                                  preferred_element_type=jnp.float32)
    r_ref[1, 0] = lax.dot_general(y, y, (((1,), (1,)), ((), ())),
                                  preferred_element_type=jnp.float32)


def _outer(z, y1):
    return pl.pallas_call(
        _outer_body,
        grid=(T,),
        in_specs=[
            pl.BlockSpec((1, N, H2), lambda i: (i, 0, 0)),
            pl.BlockSpec((1, N, H2), lambda i: (i, 0, 0)),
        ],
        out_specs=pl.BlockSpec((2, 1, N, N), lambda i: (0, i, 0, 0)),
        out_shape=jax.ShapeDtypeStruct((2, T, N, N), jnp.float32),
    )(z, y1)


def kernel(struct_features, edge_index, edge_weight, noise, W1, W2,
           tcn_w0, tcn_b0, tcn_w1, tcn_b1):
    edge_index = edge_index.astype(jnp.int32)
    edge_weight = edge_weight.astype(jnp.float32)

    adj = _build_adj(edge_index, edge_weight).reshape(T, N, N)

    z = _gcn(adj, struct_features, W1, noise, W2)
    s = z.reshape(T, NH)

    y1 = _tcn(s, tcn_w0.reshape(2 * NH, NH), tcn_w1.reshape(2 * NH, NH),
              tcn_b0.reshape(1, NH), tcn_b1.reshape(1, NH))

    out = _outer(z, y1.reshape(T, N, H2))
    return out.reshape(2, T, N * N)


# speedup vs baseline: 1.0298x; 1.0157x over previous
import functools

import jax
import jax.numpy as jnp
from jax import lax
from jax.experimental import pallas as pl
from jax.experimental.pallas import tpu as pltpu
from jax.experimental.pallas import tpu_sc as plsc

T = 8
N = 512
F = 128
H1 = 64
H2 = 8
E = 8192
NH = N * H2

_NC = 2
_NS = 16
_TS_PER_CORE = T // _NC
_CHUNKS = _NS // _TS_PER_CORE
_EC = E // _CHUNKS
_A_WORDS = N * N
_SH_WORDS = _TS_PER_CORE * _A_WORDS
_ZW = _SH_WORDS // _NS
_SCAT = 128
_NSCAT = _EC // _SCAT
_NIDX = _EC // 16


_ZB = 8192


def _adj_body(edges, weights, out, src_v, dst_v, w_v, idx2, w2, zbuf,
              sem_e, sem_z, a_sh):
    c = lax.axis_index("c")
    s = lax.axis_index("s")
    ts_local = s // _CHUNKS
    ts = c * _TS_PER_CORE + ts_local
    chunk = s % _CHUNKS

    ec = [
        pltpu.async_copy(edges.at[ts, 0, pl.ds(chunk * _EC, _EC)], src_v,
                         sem_e),
        pltpu.async_copy(edges.at[ts, 1, pl.ds(chunk * _EC, _EC)], dst_v,
                         sem_e),
        pltpu.async_copy(weights.at[ts, pl.ds(chunk * _EC, _EC)], w_v,
                         sem_e),
    ]

    def zfill(i, _):
        zbuf[pl.ds(i * 16, 16)] = jnp.zeros((16,), jnp.float32)
        return 0

    lax.fori_loop(0, _ZB // 16, zfill, 0)

    zc = [pltpu.async_copy(zbuf, a_sh.at[pl.ds(s * _ZW + i * _ZB, _ZB)],
                           sem_z)
          for i in range(_ZW // _ZB)]
    for copy in ec:
        copy.wait()

    base = ts_local * _A_WORDS

    def idx_body(i, _):
        d = dst_v[pl.ds(i * 16, 16)]
        sr = src_v[pl.ds(i * 16, 16)]
        ww = w_v[pl.ds(i * 16, 16)]
        ix = base + d * N + sr
        row = i // (_SCAT // 16)
        col = (i % (_SCAT // 16)) * 16
        idx2[row, pl.ds(col, 16)] = ix
        w2[row, pl.ds(col, 16)] = ww
        return 0

    lax.fori_loop(0, _NIDX, idx_body, 0)
    for copy in zc:
        copy.wait()
    plsc.subcore_barrier()

    def scat_body(j, _):
        pltpu.sync_copy(w2.at[j], a_sh.at[idx2.at[j]], add=True)
        return 0

    lax.fori_loop(0, _NSCAT, scat_body, 0)
    plsc.subcore_barrier()

    out_row = c * _TS_PER_CORE + s // _CHUNKS
    out_col = (s % _CHUNKS) * _ZW
    pltpu.sync_copy(a_sh.at[pl.ds(s * _ZW, _ZW)],
                    out.at[out_row, pl.ds(out_col, _ZW)])


def _build_adj(edge_index, edge_weight):
    mesh = plsc.VectorSubcoreMesh(core_axis_name="c", subcore_axis_name="s")
    k = pl.kernel(
        _adj_body,
        mesh=mesh,
        out_type=jax.ShapeDtypeStruct((T, _A_WORDS), jnp.float32),
        scratch_types=[
            pltpu.VMEM((_EC,), jnp.int32),
            pltpu.VMEM((_EC,), jnp.int32),
            pltpu.VMEM((_EC,), jnp.float32),
            pltpu.VMEM((_NSCAT, _SCAT), jnp.int32),
            pltpu.VMEM((_NSCAT, _SCAT), jnp.float32),
            pltpu.VMEM((_ZB,), jnp.float32),
            pltpu.SemaphoreType.DMA,
            pltpu.SemaphoreType.DMA,
            pltpu.VMEM_SHARED((_SH_WORDS,), jnp.float32),
        ],
    )
    return k(edge_index, edge_weight)


def _xw1_body(x_ref, w1_ref, o_ref):
    o_ref[0] = jnp.dot(x_ref[0], w1_ref[0], preferred_element_type=jnp.float32)


def _xw1(x, W1):
    return pl.pallas_call(
        _xw1_body,
        grid=(T,),
        in_specs=[
            pl.BlockSpec((1, N, F), lambda i: (i, 0, 0)),
            pl.BlockSpec((1, F, H1), lambda i: (i, 0, 0)),
        ],
        out_specs=pl.BlockSpec((1, N, H1), lambda i: (i, 0, 0)),
        out_shape=jax.ShapeDtypeStruct((T, N, H1), jnp.float32),
    )(x, W1)


def _gcn_body(a_ref, x_ref, w1_ref, n_ref, w2_ref, z_ref):
    a = a_ref[0]
    xw1 = jnp.dot(x_ref[0], w1_ref[0], preferred_element_type=jnp.float32)
    h = jnp.maximum(jnp.dot(a, xw1, preferred_element_type=jnp.float32),
                    0.0)
    h = h + 0.1 * n_ref[0]
    hw2 = jnp.dot(h, w2_ref[0], preferred_element_type=jnp.float32)
    z_ref[0] = jnp.dot(a, hw2, preferred_element_type=jnp.float32)


def _gcn(adj, x, W1, noise, W2):
    return pl.pallas_call(
        _gcn_body,
        grid=(T,),
        in_specs=[
            pl.BlockSpec((1, N, N), lambda i: (i, 0, 0)),
            pl.BlockSpec((1, N, F), lambda i: (i, 0, 0)),
            pl.BlockSpec((1, F, H1), lambda i: (i, 0, 0)),
            pl.BlockSpec((1, N, H1), lambda i: (i, 0, 0)),
            pl.BlockSpec((1, H1, H2), lambda i: (i, 0, 0)),
        ],
        out_specs=pl.BlockSpec((1, N, H2), lambda i: (i, 0, 0)),
        out_shape=jax.ShapeDtypeStruct((T, N, H2), jnp.float32),
    )(adj, x, W1, noise, W2)


_KB = 512
_NKB = 2 * NH // _KB


def _shifted(x, shift):
    return jnp.concatenate(
        [jnp.zeros((shift, NH), jnp.float32), x[:T - shift]], 0)


def _tcn_body(s_ref, w0_ref, w1_ref, b0_ref, b1_ref, y1_ref, x2_scr, acc):
    st = pl.program_id(0)
    k = pl.program_id(1)

    @pl.when((st == 0) & (k == 0))
    def _init0():
        s = s_ref[...]
        x2_scr[:, pl.ds(0, NH)] = _shifted(s, 1)
        x2_scr[:, pl.ds(NH, NH)] = s
        acc[...] = jnp.broadcast_to(b0_ref[...], (T, NH))

    @pl.when(st == 0)
    def _acc0():
        xb = x2_scr[:, pl.ds(k * _KB, _KB)]
        acc[...] += jnp.dot(xb, w0_ref[...],
                            preferred_element_type=jnp.float32)

    @pl.when(st == 1)
    def _acc1():
        xb = x2_scr[:, pl.ds(k * _KB, _KB)]
        acc[...] += jnp.dot(xb, w1_ref[...],
                            preferred_element_type=jnp.float32)

    @pl.when((st == 0) & (k == _NKB - 1))
    def _fin0():
        y0 = jnp.maximum(jnp.maximum(acc[...], 0.0) + s_ref[...], 0.0)
        x2_scr[:, pl.ds(0, NH)] = _shifted(y0, 2)
        x2_scr[:, pl.ds(NH, NH)] = y0
        acc[...] = jnp.broadcast_to(b1_ref[...], (T, NH))

    @pl.when((st == 1) & (k == _NKB - 1))
    def _fin1():
        y0 = x2_scr[:, pl.ds(NH, NH)]
        y1_ref[...] = jnp.maximum(jnp.maximum(acc[...], 0.0) + y0, 0.0)


def _tcn(s, w0_cat, w1_cat, b0, b1):
    return pl.pallas_call(
        _tcn_body,
        grid=(2, _NKB),
        in_specs=[
            pl.BlockSpec((T, NH), lambda st, k: (0, 0)),
            pl.BlockSpec((_KB, NH),
                         lambda st, k: (jnp.where(st == 0, k, _NKB - 1), 0)),
            pl.BlockSpec((_KB, NH),
                         lambda st, k: (jnp.where(st == 0, 0, k), 0)),
            pl.BlockSpec((1, NH), lambda st, k: (0, 0)),
            pl.BlockSpec((1, NH), lambda st, k: (0, 0)),
        ],
        out_specs=pl.BlockSpec((T, NH), lambda st, k: (0, 0)),
        out_shape=jax.ShapeDtypeStruct((T, NH), jnp.float32),
        scratch_shapes=[
            pltpu.VMEM((T, 2 * NH), jnp.float32),
            pltpu.VMEM((T, NH), jnp.float32),
        ],
    )(s, w0_cat, w1_cat, b0, b1)


def _outer_body(z_ref, y_ref, r_ref):
    z = z_ref[0]
    y = y_ref[0]
    r_ref[0, 0] = lax.dot_general(z, z, (((1,), (1,)), ((), ())),
                                  preferred_element_type=jnp.float32)
    r_ref[1, 0] = lax.dot_general(y, y, (((1,), (1,)), ((), ())),
                                  preferred_element_type=jnp.float32)


def _outer(z, y1):
    return pl.pallas_call(
        _outer_body,
        grid=(T,),
        in_specs=[
            pl.BlockSpec((1, N, H2), lambda i: (i, 0, 0)),
            pl.BlockSpec((1, N, H2), lambda i: (i, 0, 0)),
        ],
        out_specs=pl.BlockSpec((2, 1, N, N), lambda i: (0, i, 0, 0)),
        out_shape=jax.ShapeDtypeStruct((2, T, N, N), jnp.float32),
    )(z, y1)


def kernel(struct_features, edge_index, edge_weight, noise, W1, W2,
           tcn_w0, tcn_b0, tcn_w1, tcn_b1):
    edge_index = edge_index.astype(jnp.int32)
    edge_weight = edge_weight.astype(jnp.float32)

    adj = _build_adj(edge_index, edge_weight).reshape(T, N, N)

    z = _gcn(adj, struct_features, W1, noise, W2)
    s = z.reshape(T, NH)

    y1 = _tcn(s, tcn_w0.reshape(2 * NH, NH), tcn_w1.reshape(2 * NH, NH),
              tcn_b0.reshape(1, NH), tcn_b1.reshape(1, NH))

    out = _outer(z, y1.reshape(T, N, H2))
    return out.reshape(2, T, N * N)


# speedup vs baseline: 1.0368x; 1.0067x over previous
import functools

import jax
import jax.numpy as jnp
from jax import lax
from jax.experimental import pallas as pl
from jax.experimental.pallas import tpu as pltpu
from jax.experimental.pallas import tpu_sc as plsc

T = 8
N = 512
F = 128
H1 = 64
H2 = 8
E = 8192
NH = N * H2

_NC = 2
_NS = 16
_TS_PER_CORE = T // _NC
_CHUNKS = _NS // _TS_PER_CORE
_EC = E // _CHUNKS
_A_WORDS = N * N
_SH_WORDS = _TS_PER_CORE * _A_WORDS
_ZW = _SH_WORDS // _NS
_SCAT = 128
_NSCAT = _EC // _SCAT
_NIDX = _EC // 16


_ZB = 8192


def _adj_body(edges, weights, out, src_v, dst_v, w_v, idx2, w2, zbuf,
              sem_e, sem_z, a_sh):
    c = lax.axis_index("c")
    s = lax.axis_index("s")
    ts_local = s // _CHUNKS
    ts = c * _TS_PER_CORE + ts_local
    chunk = s % _CHUNKS

    ec = [
        pltpu.async_copy(edges.at[ts, 0, pl.ds(chunk * _EC, _EC)], src_v,
                         sem_e),
        pltpu.async_copy(edges.at[ts, 1, pl.ds(chunk * _EC, _EC)], dst_v,
                         sem_e),
        pltpu.async_copy(weights.at[ts, pl.ds(chunk * _EC, _EC)], w_v,
                         sem_e),
    ]

    def zfill(i, _):
        zbuf[pl.ds(i * 16, 16)] = jnp.zeros((16,), jnp.float32)
        return 0

    lax.fori_loop(0, _ZB // 16, zfill, 0)

    zc = [pltpu.async_copy(zbuf, a_sh.at[pl.ds(s * _ZW + i * _ZB, _ZB)],
                           sem_z)
          for i in range(_ZW // _ZB)]
    for copy in ec:
        copy.wait()

    base = ts_local * _A_WORDS

    def idx_body(i, _):
        d = dst_v[pl.ds(i * 16, 16)]
        sr = src_v[pl.ds(i * 16, 16)]
        ww = w_v[pl.ds(i * 16, 16)]
        ix = base + d * N + sr
        row = i // (_SCAT // 16)
        col = (i % (_SCAT // 16)) * 16
        idx2[row, pl.ds(col, 16)] = ix
        w2[row, pl.ds(col, 16)] = ww
        return 0

    lax.fori_loop(0, _NIDX, idx_body, 0)
    for copy in zc:
        copy.wait()
    plsc.subcore_barrier()

    sc = [pltpu.async_copy(w2.at[j], a_sh.at[idx2.at[j]], sem_z, add=True)
          for j in range(_NSCAT)]
    for copy in sc:
        copy.wait()
    plsc.subcore_barrier()

    out_row = c * _TS_PER_CORE + s // _CHUNKS
    out_col = (s % _CHUNKS) * _ZW
    pltpu.sync_copy(a_sh.at[pl.ds(s * _ZW, _ZW)],
                    out.at[out_row, pl.ds(out_col, _ZW)])


def _build_adj(edge_index, edge_weight):
    mesh = plsc.VectorSubcoreMesh(core_axis_name="c", subcore_axis_name="s")
    k = pl.kernel(
        _adj_body,
        mesh=mesh,
        out_type=jax.ShapeDtypeStruct((T, _A_WORDS), jnp.float32),
        scratch_types=[
            pltpu.VMEM((_EC,), jnp.int32),
            pltpu.VMEM((_EC,), jnp.int32),
            pltpu.VMEM((_EC,), jnp.float32),
            pltpu.VMEM((_NSCAT, _SCAT), jnp.int32),
            pltpu.VMEM((_NSCAT, _SCAT), jnp.float32),
            pltpu.VMEM((_ZB,), jnp.float32),
            pltpu.SemaphoreType.DMA,
            pltpu.SemaphoreType.DMA,
            pltpu.VMEM_SHARED((_SH_WORDS,), jnp.float32),
        ],
    )
    return k(edge_index, edge_weight)


def _xw1_body(x_ref, w1_ref, o_ref):
    o_ref[0] = jnp.dot(x_ref[0], w1_ref[0], preferred_element_type=jnp.float32)


def _xw1(x, W1):
    return pl.pallas_call(
        _xw1_body,
        grid=(T,),
        in_specs=[
            pl.BlockSpec((1, N, F), lambda i: (i, 0, 0)),
            pl.BlockSpec((1, F, H1), lambda i: (i, 0, 0)),
        ],
        out_specs=pl.BlockSpec((1, N, H1), lambda i: (i, 0, 0)),
        out_shape=jax.ShapeDtypeStruct((T, N, H1), jnp.float32),
    )(x, W1)


def _gcn_body(a_ref, x_ref, w1_ref, n_ref, w2_ref, z_ref):
    a = a_ref[0]
    xw1 = jnp.dot(x_ref[0], w1_ref[0], preferred_element_type=jnp.float32)
    h = jnp.maximum(jnp.dot(a, xw1, preferred_element_type=jnp.float32),
                    0.0)
    h = h + 0.1 * n_ref[0]
    hw2 = jnp.dot(h, w2_ref[0], preferred_element_type=jnp.float32)
    z_ref[0] = jnp.dot(a, hw2, preferred_element_type=jnp.float32)


def _gcn(adj, x, W1, noise, W2):
    return pl.pallas_call(
        _gcn_body,
        grid=(T,),
        in_specs=[
            pl.BlockSpec((1, N, N), lambda i: (i, 0, 0)),
            pl.BlockSpec((1, N, F), lambda i: (i, 0, 0)),
            pl.BlockSpec((1, F, H1), lambda i: (i, 0, 0)),
            pl.BlockSpec((1, N, H1), lambda i: (i, 0, 0)),
            pl.BlockSpec((1, H1, H2), lambda i: (i, 0, 0)),
        ],
        out_specs=pl.BlockSpec((1, N, H2), lambda i: (i, 0, 0)),
        out_shape=jax.ShapeDtypeStruct((T, N, H2), jnp.float32),
    )(adj, x, W1, noise, W2)


_KB = 512
_NKB = 2 * NH // _KB


def _shifted(x, shift):
    return jnp.concatenate(
        [jnp.zeros((shift, NH), jnp.float32), x[:T - shift]], 0)


def _tcn_body(s_ref, w0_ref, w1_ref, b0_ref, b1_ref, y1_ref, x2_scr, acc):
    st = pl.program_id(0)
    k = pl.program_id(1)

    @pl.when((st == 0) & (k == 0))
    def _init0():
        s = s_ref[...]
        x2_scr[:, pl.ds(0, NH)] = _shifted(s, 1)
        x2_scr[:, pl.ds(NH, NH)] = s
        acc[...] = jnp.broadcast_to(b0_ref[...], (T, NH))

    @pl.when(st == 0)
    def _acc0():
        xb = x2_scr[:, pl.ds(k * _KB, _KB)]
        acc[...] += jnp.dot(xb, w0_ref[...],
                            preferred_element_type=jnp.float32)

    @pl.when(st == 1)
    def _acc1():
        xb = x2_scr[:, pl.ds(k * _KB, _KB)]
        acc[...] += jnp.dot(xb, w1_ref[...],
                            preferred_element_type=jnp.float32)

    @pl.when((st == 0) & (k == _NKB - 1))
    def _fin0():
        y0 = jnp.maximum(jnp.maximum(acc[...], 0.0) + s_ref[...], 0.0)
        x2_scr[:, pl.ds(0, NH)] = _shifted(y0, 2)
        x2_scr[:, pl.ds(NH, NH)] = y0
        acc[...] = jnp.broadcast_to(b1_ref[...], (T, NH))

    @pl.when((st == 1) & (k == _NKB - 1))
    def _fin1():
        y0 = x2_scr[:, pl.ds(NH, NH)]
        y1_ref[...] = jnp.maximum(jnp.maximum(acc[...], 0.0) + y0, 0.0)


def _tcn(s, w0_cat, w1_cat, b0, b1):
    return pl.pallas_call(
        _tcn_body,
        grid=(2, _NKB),
        in_specs=[
            pl.BlockSpec((T, NH), lambda st, k: (0, 0)),
            pl.BlockSpec((_KB, NH),
                         lambda st, k: (jnp.where(st == 0, k, _NKB - 1), 0)),
            pl.BlockSpec((_KB, NH),
                         lambda st, k: (jnp.where(st == 0, 0, k), 0)),
            pl.BlockSpec((1, NH), lambda st, k: (0, 0)),
            pl.BlockSpec((1, NH), lambda st, k: (0, 0)),
        ],
        out_specs=pl.BlockSpec((T, NH), lambda st, k: (0, 0)),
        out_shape=jax.ShapeDtypeStruct((T, NH), jnp.float32),
        scratch_shapes=[
            pltpu.VMEM((T, 2 * NH), jnp.float32),
            pltpu.VMEM((T, NH), jnp.float32),
        ],
    )(s, w0_cat, w1_cat, b0, b1)


def _outer_body(z_ref, y_ref, r_ref):
    z = z_ref[0]
    y = y_ref[0]
    r_ref[0, 0] = lax.dot_general(z, z, (((1,), (1,)), ((), ())),
                                  preferred_element_type=jnp.float32)
    r_ref[1, 0] = lax.dot_general(y, y, (((1,), (1,)), ((), ())),
                                  preferred_element_type=jnp.float32)


def _outer(z, y1):
    return pl.pallas_call(
        _outer_body,
        grid=(T,),
        in_specs=[
            pl.BlockSpec((1, N, H2), lambda i: (i, 0, 0)),
            pl.BlockSpec((1, N, H2), lambda i: (i, 0, 0)),
        ],
        out_specs=pl.BlockSpec((2, 1, N, N), lambda i: (0, i, 0, 0)),
        out_shape=jax.ShapeDtypeStruct((2, T, N, N), jnp.float32),
    )(z, y1)


def kernel(struct_features, edge_index, edge_weight, noise, W1, W2,
           tcn_w0, tcn_b0, tcn_w1, tcn_b1):
    edge_index = edge_index.astype(jnp.int32)
    edge_weight = edge_weight.astype(jnp.float32)

    adj = _build_adj(edge_index, edge_weight).reshape(T, N, N)

    z = _gcn(adj, struct_features, W1, noise, W2)
    s = z.reshape(T, NH)

    y1 = _tcn(s, tcn_w0.reshape(2 * NH, NH), tcn_w1.reshape(2 * NH, NH),
              tcn_b0.reshape(1, NH), tcn_b1.reshape(1, NH))

    out = _outer(z, y1.reshape(T, N, H2))
    return out.reshape(2, T, N * N)
